# Initial kernel scaffold; baseline (speedup 1.0000x reference)
import jax
import jax.numpy as jnp
from jax.experimental import pallas as pl


def kernel(x, adj, W1, a_src1, a_dst1, W2, a_src2, a_dst2):
    raise NotImplementedError("write your pallas kernel here")


# speedup vs baseline: 15.2854x; 15.2854x over previous
import functools

import jax
import jax.numpy as jnp
from jax import lax
from jax.experimental import pallas as pl
from jax.experimental.pallas import tpu as pltpu
from jax.experimental.pallas import tpu_sc as plsc

N = 10000
D = 128
E = 320000

NC = 2
NS = 16
NW = NC * NS
EPW = E // NW
B = 80
NBLK = EPW // B
RPT = N // NS


def _prep_body(x_ref, w_ref, as_ref, ad_ref, h_ref, s_ref, d_ref):
    h = jnp.dot(x_ref[...], w_ref[...], preferred_element_type=jnp.float32)
    h_ref[...] = h
    s_ref[...] = jnp.sum(h * as_ref[...][None, :], axis=1, keepdims=True)
    d_ref[...] = jnp.sum(h * ad_ref[...][None, :], axis=1, keepdims=True)


def _tc_prep(x, W, a_s, a_d):
    return pl.pallas_call(
        _prep_body,
        out_shape=[
            jax.ShapeDtypeStruct((N, D), jnp.float32),
            jax.ShapeDtypeStruct((N, 1), jnp.float32),
            jax.ShapeDtypeStruct((N, 1), jnp.float32),
        ],
    )(x, W, a_s, a_d)


def _epilogue(numer_ref, den_ref):
    agg = numer_ref[0] + numer_ref[1]
    den = jnp.sum(den_ref[...], axis=(0, 2)) + 1e-9
    y = jnp.maximum(agg / den[:, None], 0.0)
    nrm = jnp.sqrt(jnp.sum(y * y, axis=1, keepdims=True))
    return y / jnp.maximum(nrm, 1e-6)


def _mid_body(numer_ref, den_ref, w_ref, as_ref, ad_ref, h_ref, s_ref, d_ref):
    x2 = _epilogue(numer_ref, den_ref)
    h = jnp.dot(x2, w_ref[...], preferred_element_type=jnp.float32)
    h_ref[...] = h
    s_ref[...] = jnp.sum(h * as_ref[...][None, :], axis=1, keepdims=True)
    d_ref[...] = jnp.sum(h * ad_ref[...][None, :], axis=1, keepdims=True)


def _tc_mid(numer_p, den_p, W, a_s, a_d):
    return pl.pallas_call(
        _mid_body,
        out_shape=[
            jax.ShapeDtypeStruct((N, D), jnp.float32),
            jax.ShapeDtypeStruct((N, 1), jnp.float32),
            jax.ShapeDtypeStruct((N, 1), jnp.float32),
        ],
    )(numer_p, den_p, W, a_s, a_d)


def _final_body(numer_ref, den_ref, out_ref):
    out_ref[...] = _epilogue(numer_ref, den_ref)


def _tc_final(numer_p, den_p):
    return pl.pallas_call(
        _final_body,
        out_shape=jax.ShapeDtypeStruct((N, D), jnp.float32),
    )(numer_p, den_p)


_MESH = plsc.VectorSubcoreMesh(core_axis_name="c", subcore_axis_name="s")


@functools.partial(
    pl.kernel,
    out_type=[
        jax.ShapeDtypeStruct((NC, N, D), jnp.float32),
        jax.ShapeDtypeStruct((NC, N, 16), jnp.float32),
    ],
    mesh=_MESH,
    scratch_types=[
        pltpu.VMEM((B,), jnp.int32),
        pltpu.VMEM((B,), jnp.int32),
        pltpu.VMEM((B, D), jnp.float32),
        pltpu.VMEM((B, 16), jnp.float32),
        pltpu.VMEM((N,), jnp.float32),
        pltpu.VMEM((N,), jnp.float32),
        pltpu.VMEM_SHARED((N, D), jnp.float32),
        pltpu.VMEM_SHARED((N, 16), jnp.float32),
        pltpu.SemaphoreType.DMA,
    ],
    compiler_params=pltpu.CompilerParams(needs_layout_passes=False,
                                         use_tc_tiling_on_sc=False),
)
def _sc_edge_pass(src_hbm, dst_hbm, s_hbm, d_hbm, h_hbm,
                  numer_out, den_out,
                  idx_src, idx_dst, rows, w_pad, s_tab, d_tab,
                  numer_sh, den_sh, sem):
    cid = lax.axis_index("c")
    sid = lax.axis_index("s")
    gwid = cid * NS + sid
    zv = jnp.zeros((16,), jnp.float32)

    def zrow(i, _):
        for j in range(D // 16):
            rows[i, pl.ds(j * 16, 16)] = zv
        w_pad[i, pl.ds(0, 16)] = zv
        return 0
    lax.fori_loop(0, B, zrow, 0)

    nchunk = N // B
    def zchunk(k, _):
        c = sid + k * NS

        @pl.when(c < nchunk)
        def _():
            pltpu.sync_copy(rows, numer_sh.at[pl.ds(c * B, B)])
            pltpu.sync_copy(w_pad, den_sh.at[pl.ds(c * B, B)])
        return 0
    lax.fori_loop(0, (nchunk + NS - 1) // NS, zchunk, 0)

    pltpu.sync_copy(s_hbm, s_tab)
    pltpu.sync_copy(d_hbm, d_tab)

    plsc.subcore_barrier()

    base = gwid * EPW
    col0 = jnp.zeros((16,), jnp.int32)

    def blk(b, _):
        eb = base + b * B
        pltpu.sync_copy(src_hbm.at[pl.ds(eb, B)], idx_src)
        pltpu.sync_copy(dst_hbm.at[pl.ds(eb, B)], idx_dst)
        pltpu.async_copy(h_hbm.at[idx_src], rows, sem).wait()
        for j in range(B // 16):
            iv = idx_src[pl.ds(j * 16, 16)]
            dv = idx_dst[pl.ds(j * 16, 16)]
            e = plsc.load_gather(s_tab, [iv]) + plsc.load_gather(d_tab, [dv])
            e = jnp.where(e >= 0, e, 0.2 * e)
            w = jnp.exp(e)
            rid = lax.iota(jnp.int32, 16) + j * 16
            plsc.store_scatter(w_pad, [rid, col0], w)
            for k in range(16):
                wi = w[k]
                r = j * 16 + k
                for c in range(D // 16):
                    rows[r, pl.ds(c * 16, 16)] = rows[r, pl.ds(c * 16, 16)] * wi

        pltpu.sync_copy(rows, numer_sh.at[idx_dst], add=True)
        pltpu.sync_copy(w_pad, den_sh.at[idx_dst], add=True)
        return 0

    lax.fori_loop(0, NBLK, blk, 0)

    plsc.subcore_barrier()

    WB = 2000

    @pl.when(sid < N // WB)
    def _():
        r0 = sid * WB
        pltpu.sync_copy(numer_sh.at[pl.ds(r0, WB)],
                        numer_out.at[cid, pl.ds(r0, WB)])
        pltpu.sync_copy(den_sh.at[pl.ds(r0, WB)],
                        den_out.at[cid, pl.ds(r0, WB)])


def kernel(x, adj, W1, a_src1, a_dst1, W2, a_src2, a_dst2):
    src1, dst1 = adj[0, 0], adj[0, 1]
    src2, dst2 = adj[1, 0], adj[1, 1]

    h1, s1, d1 = _tc_prep(x, W1, a_src1, a_dst1)
    numer1, den1 = _sc_edge_pass(src1, dst1, s1.reshape(N), d1.reshape(N), h1)
    h2, s2, d2 = _tc_mid(numer1, den1, W2, a_src2, a_dst2)
    numer2, den2 = _sc_edge_pass(src2, dst2, s2.reshape(N), d2.reshape(N), h2)
    return _tc_final(numer2, den2)


# speedup vs baseline: 24.3097x; 1.5904x over previous
import functools

import jax
import jax.numpy as jnp
from jax import lax
from jax.experimental import pallas as pl
from jax.experimental.pallas import tpu as pltpu
from jax.experimental.pallas import tpu_sc as plsc

N = 10000
D = 128
E = 320000

NC = 2
NS = 16
NW = NC * NS
EPW = E // NW
B = 80
NBLK = EPW // B


def _prep_body(x_ref, w_ref, as_ref, ad_ref, h_ref, s_ref, d_ref):
    h = jnp.dot(x_ref[...], w_ref[...], preferred_element_type=jnp.float32)
    h_ref[...] = h
    s_ref[...] = jnp.sum(h * as_ref[...][None, :], axis=1, keepdims=True)
    d_ref[...] = jnp.sum(h * ad_ref[...][None, :], axis=1, keepdims=True)


def _tc_prep(x, W, a_s, a_d):
    return pl.pallas_call(
        _prep_body,
        out_shape=[
            jax.ShapeDtypeStruct((N, D), jnp.float32),
            jax.ShapeDtypeStruct((N, 1), jnp.float32),
            jax.ShapeDtypeStruct((N, 1), jnp.float32),
        ],
    )(x, W, a_s, a_d)


def _epilogue(numer_ref, den_ref):
    agg = numer_ref[0] + numer_ref[1]
    den = jnp.sum(den_ref[...], axis=0) + 1e-9
    y = jnp.maximum(agg / den[:, None], 0.0)
    nrm = jnp.sqrt(jnp.sum(y * y, axis=1, keepdims=True))
    return y / jnp.maximum(nrm, 1e-6)


def _mid_body(numer_ref, den_ref, w_ref, as_ref, ad_ref, h_ref, s_ref, d_ref):
    x2 = _epilogue(numer_ref, den_ref)
    h = jnp.dot(x2, w_ref[...], preferred_element_type=jnp.float32)
    h_ref[...] = h
    s_ref[...] = jnp.sum(h * as_ref[...][None, :], axis=1, keepdims=True)
    d_ref[...] = jnp.sum(h * ad_ref[...][None, :], axis=1, keepdims=True)


def _tc_mid(numer_p, den_p, W, a_s, a_d):
    return pl.pallas_call(
        _mid_body,
        out_shape=[
            jax.ShapeDtypeStruct((N, D), jnp.float32),
            jax.ShapeDtypeStruct((N, 1), jnp.float32),
            jax.ShapeDtypeStruct((N, 1), jnp.float32),
        ],
    )(numer_p, den_p, W, a_s, a_d)


def _final_body(numer_ref, den_ref, out_ref):
    out_ref[...] = _epilogue(numer_ref, den_ref)


def _tc_final(numer_p, den_p):
    return pl.pallas_call(
        _final_body,
        out_shape=jax.ShapeDtypeStruct((N, D), jnp.float32),
    )(numer_p, den_p)


_MESH = plsc.VectorSubcoreMesh(core_axis_name="c", subcore_axis_name="s")


@functools.partial(
    pl.kernel,
    out_type=[
        jax.ShapeDtypeStruct((NC, N, D), jnp.float32),
        jax.ShapeDtypeStruct((NW, N), jnp.float32),
    ],
    mesh=_MESH,
    scratch_types=[
        pltpu.VMEM((4, B), jnp.int32),
        pltpu.VMEM((2, B), jnp.int32),
        pltpu.VMEM((B, D), jnp.float32),
        pltpu.VMEM((B, D), jnp.float32),
        pltpu.VMEM((N,), jnp.float32),
        pltpu.VMEM((N,), jnp.float32),
        pltpu.VMEM((N,), jnp.float32),
        pltpu.VMEM_SHARED((N, D), jnp.float32),
        pltpu.SemaphoreType.DMA,
        pltpu.SemaphoreType.DMA,
        pltpu.SemaphoreType.DMA,
        pltpu.SemaphoreType.DMA,
        pltpu.SemaphoreType.DMA,
        pltpu.SemaphoreType.DMA,
        pltpu.SemaphoreType.DMA,
        pltpu.SemaphoreType.DMA,
        pltpu.SemaphoreType.DMA,
        pltpu.SemaphoreType.DMA,
    ],
    compiler_params=pltpu.CompilerParams(needs_layout_passes=False,
                                         use_tc_tiling_on_sc=False),
)
def _sc_edge_pass(adj_hbm, s_hbm, d_hbm, h_hbm,
                  numer_out, den_out,
                  sring, dbuf, rows0, rows1, s_tab, d_tab, den_loc,
                  numer_sh,
                  gsem0, gsem1, nsem0, nsem1,
                  isem0, isem1, isem2, isem3, jsem0, jsem1):
    cid = lax.axis_index("c")
    sid = lax.axis_index("s")
    gwid = cid * NS + sid
    zv = jnp.zeros((16,), jnp.float32)
    rows = (rows0, rows1)
    gsem = (gsem0, gsem1)
    nsem = (nsem0, nsem1)
    isem = (isem0, isem1, isem2, isem3)
    jsem = (jsem0, jsem1)

    def zrow(i, _):
        for j in range(D // 16):
            rows0[i, pl.ds(j * 16, 16)] = zv
        return 0
    lax.fori_loop(0, B, zrow, 0)

    def zden(i, _):
        den_loc[pl.ds(i * 16, 16)] = zv
        return 0
    lax.fori_loop(0, N // 16, zden, 0)

    nchunk = N // B
    def zchunk(k, _):
        c = sid + k * NS

        @pl.when(c < nchunk)
        def _():
            pltpu.sync_copy(rows0, numer_sh.at[pl.ds(c * B, B)])
        return 0
    lax.fori_loop(0, (nchunk + NS - 1) // NS, zchunk, 0)

    pltpu.sync_copy(s_hbm, s_tab)
    pltpu.sync_copy(d_hbm, d_tab)

    plsc.subcore_barrier()

    base = gwid * EPW

    def sfetch_start(blk, sl):
        eb = base + blk * B
        pltpu.async_copy(adj_hbm.at[0, pl.ds(eb, B)], sring.at[sl], isem[sl])

    def sfetch_wait(sl):
        pltpu.make_async_copy(adj_hbm.at[0, pl.ds(0, B)], sring.at[sl],
                              isem[sl]).wait()

    def dfetch_start(blk, b):
        eb = base + blk * B
        pltpu.async_copy(adj_hbm.at[1, pl.ds(eb, B)], dbuf.at[b], jsem[b])

    def dfetch_wait(b):
        pltpu.make_async_copy(adj_hbm.at[1, pl.ds(0, B)], dbuf.at[b],
                              jsem[b]).wait()

    def gather_start(sl, b):
        pltpu.async_copy(h_hbm.at[sring.at[sl]], rows[b], gsem[b])

    def gather_wait(b):
        pltpu.make_async_copy(h_hbm.at[sring.at[0]], rows[b], gsem[b]).wait()

    def scatter_start(b):
        pltpu.async_copy(rows[b], numer_sh.at[dbuf.at[b]], nsem[b],
                         add=True)

    def scatter_wait(b):
        pltpu.make_async_copy(rows[b], numer_sh.at[dbuf.at[b]],
                              nsem[b]).wait()

    def compute(sl, b):
        rb = rows[b]
        for j in range(B // 16):
            iv = sring[sl, pl.ds(j * 16, 16)]
            dv = dbuf[b, pl.ds(j * 16, 16)]
            e = plsc.load_gather(s_tab, [iv]) + plsc.load_gather(d_tab, [dv])
            e = jnp.where(e >= 0, e, 0.2 * e)
            w = jnp.exp(e)
            plsc.addupdate_scatter(den_loc, [dv], w)
            for k in range(16):
                wi = w[k]
                r = j * 16 + k
                for c in range(D // 16):
                    rb[r, pl.ds(c * 16, 16)] = rb[r, pl.ds(c * 16, 16)] * wi

    sfetch_start(0, 0)
    sfetch_start(1, 1)
    dfetch_start(0, 0)
    sfetch_wait(0)
    gather_start(0, 0)

    def quad(g, _):
        for b in range(4):
            blk = g * 4 + b
            sl = b
            rb = b % 2
            ob = 1 - rb
            sl1 = (b + 1) % 4
            sl2 = (b + 2) % 4

            @pl.when(blk < NBLK)
            def _():
                @pl.when(blk + 1 < NBLK)
                def _():
                    @pl.when(blk >= 1)
                    def _():
                        scatter_wait(ob)
                    dfetch_start(blk + 1, ob)

                    @pl.when(blk + 2 < NBLK)
                    def _():
                        sfetch_start(blk + 2, sl2)
                    sfetch_wait(sl1)
                    gather_start(sl1, ob)

                gather_wait(rb)
                dfetch_wait(rb)
                compute(sl, rb)
                scatter_start(rb)
        return 0
    lax.fori_loop(0, (NBLK + 3) // 4, quad, 0)

    scatter_wait(0)
    scatter_wait(1)

    pltpu.sync_copy(den_loc, den_out.at[gwid])

    plsc.subcore_barrier()

    WB = 2000

    @pl.when(sid < N // WB)
    def _():
        r0 = sid * WB
        pltpu.sync_copy(numer_sh.at[pl.ds(r0, WB)],
                        numer_out.at[cid, pl.ds(r0, WB)])


def kernel(x, adj, W1, a_src1, a_dst1, W2, a_src2, a_dst2):
    h1, s1, d1 = _tc_prep(x, W1, a_src1, a_dst1)
    numer1, den1 = _sc_edge_pass(adj[0], s1.reshape(N), d1.reshape(N), h1)
    h2, s2, d2 = _tc_mid(numer1, den1, W2, a_src2, a_dst2)
    numer2, den2 = _sc_edge_pass(adj[1], s2.reshape(N), d2.reshape(N), h2)
    return _tc_final(numer2, den2)


# speedup vs baseline: 26.3585x; 1.0843x over previous
import functools

import jax
import jax.numpy as jnp
from jax import lax
from jax.experimental import pallas as pl
from jax.experimental.pallas import tpu as pltpu
from jax.experimental.pallas import tpu_sc as plsc

N = 10000
D = 128
DH = D // 2
E = 320000

NC = 2
NS = 16
NW = NC * NS
EPT = E // NS
B = 80
NBLK = EPT // B
DEPTH = 6
GA = 3
FA = 4


def _prep_body(x_ref, w_ref, as_ref, ad_ref, hlo_ref, hhi_ref, s_ref, d_ref):
    h = jnp.dot(x_ref[...], w_ref[...], preferred_element_type=jnp.float32)
    hlo_ref[...] = h[:, :DH]
    hhi_ref[...] = h[:, DH:]
    s_ref[...] = jnp.sum(h * as_ref[...][None, :], axis=1, keepdims=True)
    d_ref[...] = jnp.sum(h * ad_ref[...][None, :], axis=1, keepdims=True)


def _tc_prep(x, W, a_s, a_d):
    return pl.pallas_call(
        _prep_body,
        out_shape=[
            jax.ShapeDtypeStruct((N, DH), jnp.float32),
            jax.ShapeDtypeStruct((N, DH), jnp.float32),
            jax.ShapeDtypeStruct((N, 1), jnp.float32),
            jax.ShapeDtypeStruct((N, 1), jnp.float32),
        ],
    )(x, W, a_s, a_d)


def _epilogue(numer_ref, den_ref):
    agg = jnp.concatenate([numer_ref[0], numer_ref[1]], axis=-1)
    den = 0.5 * jnp.sum(den_ref[...], axis=0) + 1e-9
    y = jnp.maximum(agg / den[:, None], 0.0)
    nrm = jnp.sqrt(jnp.sum(y * y, axis=1, keepdims=True))
    return y / jnp.maximum(nrm, 1e-6)


def _mid_body(numer_ref, den_ref, w_ref, as_ref, ad_ref,
              hlo_ref, hhi_ref, s_ref, d_ref):
    x2 = _epilogue(numer_ref, den_ref)
    h = jnp.dot(x2, w_ref[...], preferred_element_type=jnp.float32)
    hlo_ref[...] = h[:, :DH]
    hhi_ref[...] = h[:, DH:]
    s_ref[...] = jnp.sum(h * as_ref[...][None, :], axis=1, keepdims=True)
    d_ref[...] = jnp.sum(h * ad_ref[...][None, :], axis=1, keepdims=True)


def _tc_mid(numer_p, den_p, W, a_s, a_d):
    return pl.pallas_call(
        _mid_body,
        out_shape=[
            jax.ShapeDtypeStruct((N, DH), jnp.float32),
            jax.ShapeDtypeStruct((N, DH), jnp.float32),
            jax.ShapeDtypeStruct((N, 1), jnp.float32),
            jax.ShapeDtypeStruct((N, 1), jnp.float32),
        ],
    )(numer_p, den_p, W, a_s, a_d)


def _final_body(numer_ref, den_ref, out_ref):
    out_ref[...] = _epilogue(numer_ref, den_ref)


def _tc_final(numer_p, den_p):
    return pl.pallas_call(
        _final_body,
        out_shape=jax.ShapeDtypeStruct((N, D), jnp.float32),
    )(numer_p, den_p)


_MESH = plsc.VectorSubcoreMesh(core_axis_name="c", subcore_axis_name="s")


@functools.partial(
    pl.kernel,
    out_type=[
        jax.ShapeDtypeStruct((NC, N, DH), jnp.float32),
        jax.ShapeDtypeStruct((NW, N), jnp.float32),
    ],
    mesh=_MESH,
    scratch_types=[
        pltpu.VMEM((DEPTH, B), jnp.int32),
        pltpu.VMEM((DEPTH, B), jnp.int32),
        pltpu.VMEM((DEPTH, B, DH), jnp.float32),
        pltpu.VMEM((N,), jnp.float32),
        pltpu.VMEM((N,), jnp.float32),
        pltpu.VMEM((N,), jnp.float32),
        pltpu.VMEM_SHARED((N, DH), jnp.float32),
        pltpu.SemaphoreType.DMA((DEPTH,)),
        pltpu.SemaphoreType.DMA((DEPTH,)),
        pltpu.SemaphoreType.DMA((DEPTH,)),
        pltpu.SemaphoreType.DMA((DEPTH,)),
    ],
    compiler_params=pltpu.CompilerParams(needs_layout_passes=False,
                                         use_tc_tiling_on_sc=False),
)
def _sc_edge_pass(adj_hbm, s_hbm, d_hbm, hlo_hbm, hhi_hbm,
                  numer_out, den_out,
                  sring, dring, rows, s_tab, d_tab, den_loc,
                  numer_sh, gsem, nsem, isem, jsem):
    cid = lax.axis_index("c")
    sid = lax.axis_index("s")
    gwid = cid * NS + sid
    zv = jnp.zeros((16,), jnp.float32)

    def zrow(i, _):
        for j in range(DH // 16):
            rows[0, i, pl.ds(j * 16, 16)] = zv
        return 0
    lax.fori_loop(0, B, zrow, 0)

    def zden(i, _):
        den_loc[pl.ds(i * 16, 16)] = zv
        return 0
    lax.fori_loop(0, N // 16, zden, 0)

    nchunk = N // B
    def zchunk(k, _):
        c = sid + k * NS

        @pl.when(c < nchunk)
        def _():
            pltpu.sync_copy(rows.at[0], numer_sh.at[pl.ds(c * B, B)])
        return 0
    lax.fori_loop(0, (nchunk + NS - 1) // NS, zchunk, 0)

    pltpu.sync_copy(s_hbm, s_tab)
    pltpu.sync_copy(d_hbm, d_tab)

    plsc.subcore_barrier()

    base = sid * EPT

    def sfetch_start(blk, sl):
        eb = base + blk * B
        pltpu.async_copy(adj_hbm.at[0, pl.ds(eb, B)], sring.at[sl],
                         isem.at[sl])

    def sfetch_wait(sl):
        pltpu.make_async_copy(adj_hbm.at[0, pl.ds(0, B)], sring.at[sl],
                              isem.at[sl]).wait()

    def dfetch_start(blk, sl):
        eb = base + blk * B
        pltpu.async_copy(adj_hbm.at[1, pl.ds(eb, B)], dring.at[sl],
                         jsem.at[sl])

    def dfetch_wait(sl):
        pltpu.make_async_copy(adj_hbm.at[1, pl.ds(0, B)], dring.at[sl],
                              jsem.at[sl]).wait()

    def gather_start(sl):
        @pl.when(cid == 0)
        def _():
            pltpu.async_copy(hlo_hbm.at[sring.at[sl]], rows.at[sl],
                             gsem.at[sl])

        @pl.when(cid == 1)
        def _():
            pltpu.async_copy(hhi_hbm.at[sring.at[sl]], rows.at[sl],
                             gsem.at[sl])

    def gather_wait(sl):
        pltpu.make_async_copy(hlo_hbm.at[sring.at[sl]], rows.at[sl],
                              gsem.at[sl]).wait()

    def scatter_start(sl):
        pltpu.async_copy(rows.at[sl], numer_sh.at[dring.at[sl]], nsem.at[sl],
                         add=True)

    def scatter_wait(sl):
        pltpu.make_async_copy(rows.at[sl], numer_sh.at[dring.at[sl]],
                              nsem.at[sl]).wait()

    def compute(sl):
        for j in range(B // 16):
            iv = sring[sl, pl.ds(j * 16, 16)]
            dv = dring[sl, pl.ds(j * 16, 16)]
            e = plsc.load_gather(s_tab, [iv]) + plsc.load_gather(d_tab, [dv])
            e = jnp.where(e >= 0, e, 0.2 * e)
            w = jnp.exp(e)
            plsc.addupdate_scatter(den_loc, [dv], w)
            for k in range(16):
                wi = w[k]
                r = j * 16 + k
                for c in range(DH // 16):
                    rows[sl, r, pl.ds(c * 16, 16)] = (
                        rows[sl, r, pl.ds(c * 16, 16)] * wi)

    for p in range(FA):
        sfetch_start(p, p)
        dfetch_start(p, p)
    for p in range(GA):
        sfetch_wait(p)
        gather_start(p)

    def group(g, _):
        for b in range(DEPTH):
            blk = g * DEPTH + b
            sl = b
            sl_g = (b + GA) % DEPTH
            sl_f = (b + FA) % DEPTH

            @pl.when(blk < NBLK)
            def _():
                @pl.when(jnp.logical_and(blk >= GA, blk + GA < NBLK))
                def _():
                    scatter_wait(sl_g)

                @pl.when(blk + FA < NBLK)
                def _():
                    sfetch_start(blk + FA, sl_f)
                    dfetch_start(blk + FA, sl_f)

                @pl.when(blk + GA < NBLK)
                def _():
                    sfetch_wait(sl_g)
                    gather_start(sl_g)

                gather_wait(sl)
                dfetch_wait(sl)
                compute(sl)
                scatter_start(sl)
        return 0
    lax.fori_loop(0, (NBLK + DEPTH - 1) // DEPTH, group, 0)

    for t in range(GA):
        scatter_wait((NBLK - GA + t) % DEPTH)

    pltpu.sync_copy(den_loc, den_out.at[gwid])

    plsc.subcore_barrier()

    WB = 2000

    @pl.when(sid < N // WB)
    def _():
        r0 = sid * WB
        pltpu.sync_copy(numer_sh.at[pl.ds(r0, WB)],
                        numer_out.at[cid, pl.ds(r0, WB)])


def kernel(x, adj, W1, a_src1, a_dst1, W2, a_src2, a_dst2):
    hlo1, hhi1, s1, d1 = _tc_prep(x, W1, a_src1, a_dst1)
    numer1, den1 = _sc_edge_pass(adj[0], s1.reshape(N), d1.reshape(N),
                                 hlo1, hhi1)
    hlo2, hhi2, s2, d2 = _tc_mid(numer1, den1, W2, a_src2, a_dst2)
    numer2, den2 = _sc_edge_pass(adj[1], s2.reshape(N), d2.reshape(N),
                                 hlo2, hhi2)
    return _tc_final(numer2, den2)


# speedup vs baseline: 26.4912x; 1.0050x over previous
import functools

import jax
import jax.numpy as jnp
from jax import lax
from jax.experimental import pallas as pl
from jax.experimental.pallas import tpu as pltpu
from jax.experimental.pallas import tpu_sc as plsc

N = 10000
D = 128
DH = D // 2
E = 320000

NC = 2
NS = 16
NW = NC * NS
EPT = E // NS
B = 80
NBLK = EPT // B
DEPTH = 6
GA = 3
FA = 4


def _prep_body(x_ref, w_ref, as_ref, ad_ref, hlo_ref, hhi_ref, s_ref, d_ref):
    h = jnp.dot(x_ref[...], w_ref[...], preferred_element_type=jnp.float32)
    hlo_ref[...] = h[:, :DH]
    hhi_ref[...] = h[:, DH:]
    s_ref[...] = jnp.sum(h * as_ref[...][None, :], axis=1)[None, :]
    d_ref[...] = jnp.sum(h * ad_ref[...][None, :], axis=1)[None, :]


def _tc_prep(x, W, a_s, a_d):
    return pl.pallas_call(
        _prep_body,
        out_shape=[
            jax.ShapeDtypeStruct((N, DH), jnp.float32),
            jax.ShapeDtypeStruct((N, DH), jnp.float32),
            jax.ShapeDtypeStruct((1, N), jnp.float32),
            jax.ShapeDtypeStruct((1, N), jnp.float32),
        ],
    )(x, W, a_s, a_d)


def _epilogue(numer_ref, den_ref):
    agg = jnp.concatenate([numer_ref[0], numer_ref[1]], axis=-1)
    den = 0.5 * jnp.sum(den_ref[...], axis=0) + 1e-9
    y = jnp.maximum(agg / den[:, None], 0.0)
    nrm = jnp.sqrt(jnp.sum(y * y, axis=1, keepdims=True))
    return y / jnp.maximum(nrm, 1e-6)


def _mid_body(numer_ref, den_ref, w_ref, as_ref, ad_ref,
              hlo_ref, hhi_ref, s_ref, d_ref):
    x2 = _epilogue(numer_ref, den_ref)
    h = jnp.dot(x2, w_ref[...], preferred_element_type=jnp.float32)
    hlo_ref[...] = h[:, :DH]
    hhi_ref[...] = h[:, DH:]
    s_ref[...] = jnp.sum(h * as_ref[...][None, :], axis=1)[None, :]
    d_ref[...] = jnp.sum(h * ad_ref[...][None, :], axis=1)[None, :]


def _tc_mid(numer_p, den_p, W, a_s, a_d):
    return pl.pallas_call(
        _mid_body,
        out_shape=[
            jax.ShapeDtypeStruct((N, DH), jnp.float32),
            jax.ShapeDtypeStruct((N, DH), jnp.float32),
            jax.ShapeDtypeStruct((1, N), jnp.float32),
            jax.ShapeDtypeStruct((1, N), jnp.float32),
        ],
    )(numer_p, den_p, W, a_s, a_d)


def _final_body(numer_ref, den_ref, out_ref):
    out_ref[...] = _epilogue(numer_ref, den_ref)


def _tc_final(numer_p, den_p):
    return pl.pallas_call(
        _final_body,
        out_shape=jax.ShapeDtypeStruct((N, D), jnp.float32),
    )(numer_p, den_p)


_MESH = plsc.VectorSubcoreMesh(core_axis_name="c", subcore_axis_name="s")


@functools.partial(
    pl.kernel,
    out_type=[
        jax.ShapeDtypeStruct((NC, N, DH), jnp.float32),
        jax.ShapeDtypeStruct((NW, N), jnp.float32),
    ],
    mesh=_MESH,
    scratch_types=[
        pltpu.VMEM((DEPTH, B), jnp.int32),
        pltpu.VMEM((DEPTH, B), jnp.int32),
        pltpu.VMEM((DEPTH, B, DH), jnp.float32),
        pltpu.VMEM((N,), jnp.float32),
        pltpu.VMEM((N,), jnp.float32),
        pltpu.VMEM((N,), jnp.float32),
        pltpu.VMEM_SHARED((N, DH), jnp.float32),
        pltpu.SemaphoreType.DMA((DEPTH,)),
        pltpu.SemaphoreType.DMA((DEPTH,)),
        pltpu.SemaphoreType.DMA((DEPTH,)),
        pltpu.SemaphoreType.DMA((DEPTH,)),
    ],
    compiler_params=pltpu.CompilerParams(needs_layout_passes=False,
                                         use_tc_tiling_on_sc=False),
)
def _sc_edge_pass(adj_hbm, s_hbm, d_hbm, hlo_hbm, hhi_hbm,
                  numer_out, den_out,
                  sring, dring, rows, s_tab, d_tab, den_loc,
                  numer_sh, gsem, nsem, isem, jsem):
    cid = lax.axis_index("c")
    sid = lax.axis_index("s")
    gwid = cid * NS + sid
    zv = jnp.zeros((16,), jnp.float32)

    def zrow(i, _):
        for j in range(DH // 16):
            rows[0, i, pl.ds(j * 16, 16)] = zv
        return 0
    lax.fori_loop(0, B, zrow, 0)

    def zden(i, _):
        den_loc[pl.ds(i * 16, 16)] = zv
        return 0
    lax.fori_loop(0, N // 16, zden, 0)

    nchunk = N // B
    def zchunk(k, _):
        c = sid + k * NS

        @pl.when(c < nchunk)
        def _():
            pltpu.sync_copy(rows.at[0], numer_sh.at[pl.ds(c * B, B)])
        return 0
    lax.fori_loop(0, (nchunk + NS - 1) // NS, zchunk, 0)

    pltpu.sync_copy(s_hbm.at[0], s_tab)
    pltpu.sync_copy(d_hbm.at[0], d_tab)

    plsc.subcore_barrier()

    base = sid * EPT

    def sfetch_start(blk, sl):
        eb = base + blk * B
        pltpu.async_copy(adj_hbm.at[0, pl.ds(eb, B)], sring.at[sl],
                         isem.at[sl])

    def sfetch_wait(sl):
        pltpu.make_async_copy(adj_hbm.at[0, pl.ds(0, B)], sring.at[sl],
                              isem.at[sl]).wait()

    def dfetch_start(blk, sl):
        eb = base + blk * B
        pltpu.async_copy(adj_hbm.at[1, pl.ds(eb, B)], dring.at[sl],
                         jsem.at[sl])

    def dfetch_wait(sl):
        pltpu.make_async_copy(adj_hbm.at[1, pl.ds(0, B)], dring.at[sl],
                              jsem.at[sl]).wait()

    def gather_start(sl):
        @pl.when(cid == 0)
        def _():
            pltpu.async_copy(hlo_hbm.at[sring.at[sl]], rows.at[sl],
                             gsem.at[sl])

        @pl.when(cid == 1)
        def _():
            pltpu.async_copy(hhi_hbm.at[sring.at[sl]], rows.at[sl],
                             gsem.at[sl])

    def gather_wait(sl):
        pltpu.make_async_copy(hlo_hbm.at[sring.at[sl]], rows.at[sl],
                              gsem.at[sl]).wait()

    def scatter_start(sl):
        pltpu.async_copy(rows.at[sl], numer_sh.at[dring.at[sl]], nsem.at[sl],
                         add=True)

    def scatter_wait(sl):
        pltpu.make_async_copy(rows.at[sl], numer_sh.at[dring.at[sl]],
                              nsem.at[sl]).wait()

    def compute(sl):
        for j in range(B // 16):
            iv = sring[sl, pl.ds(j * 16, 16)]
            dv = dring[sl, pl.ds(j * 16, 16)]
            e = plsc.load_gather(s_tab, [iv]) + plsc.load_gather(d_tab, [dv])
            e = jnp.where(e >= 0, e, 0.2 * e)
            w = jnp.exp(e)
            plsc.addupdate_scatter(den_loc, [dv], w)
            for k in range(16):
                wi = w[k]
                r = j * 16 + k
                for c in range(DH // 16):
                    rows[sl, r, pl.ds(c * 16, 16)] = (
                        rows[sl, r, pl.ds(c * 16, 16)] * wi)

    for p in range(FA):
        sfetch_start(p, p)
        dfetch_start(p, p)
    for p in range(GA):
        sfetch_wait(p)
        gather_start(p)

    def group(g, _):
        for b in range(DEPTH):
            blk = g * DEPTH + b
            sl = b
            sl_g = (b + GA) % DEPTH
            sl_f = (b + FA) % DEPTH

            @pl.when(blk < NBLK)
            def _():
                @pl.when(jnp.logical_and(blk >= GA, blk + GA < NBLK))
                def _():
                    scatter_wait(sl_g)

                @pl.when(blk + FA < NBLK)
                def _():
                    sfetch_start(blk + FA, sl_f)
                    dfetch_start(blk + FA, sl_f)

                @pl.when(blk + GA < NBLK)
                def _():
                    sfetch_wait(sl_g)
                    gather_start(sl_g)

                gather_wait(sl)
                dfetch_wait(sl)
                compute(sl)
                scatter_start(sl)
        return 0
    lax.fori_loop(0, (NBLK + DEPTH - 1) // DEPTH, group, 0)

    for t in range(GA):
        scatter_wait((NBLK - GA + t) % DEPTH)

    pltpu.sync_copy(den_loc, den_out.at[gwid])

    plsc.subcore_barrier()

    WB = 2000

    @pl.when(sid < N // WB)
    def _():
        r0 = sid * WB
        pltpu.sync_copy(numer_sh.at[pl.ds(r0, WB)],
                        numer_out.at[cid, pl.ds(r0, WB)])


def kernel(x, adj, W1, a_src1, a_dst1, W2, a_src2, a_dst2):
    hlo1, hhi1, s1, d1 = _tc_prep(x, W1, a_src1, a_dst1)
    numer1, den1 = _sc_edge_pass(adj[0], s1, d1, hlo1, hhi1)
    hlo2, hhi2, s2, d2 = _tc_mid(numer1, den1, W2, a_src2, a_dst2)
    numer2, den2 = _sc_edge_pass(adj[1], s2, d2, hlo2, hhi2)
    return _tc_final(numer2, den2)


# speedup vs baseline: 26.8118x; 1.0121x over previous
import functools

import jax
import jax.numpy as jnp
from jax import lax
from jax.experimental import pallas as pl
from jax.experimental.pallas import tpu as pltpu
from jax.experimental.pallas import tpu_sc as plsc

N = 10000
D = 128
DH = D // 2
E = 320000

NC = 2
NS = 16
NW = NC * NS
EPT = E // NS
B = 80
NBLK = EPT // B
DEPTH = 6
GA = 3
FA = 4


def _prep_body(x_ref, w_ref, as_ref, ad_ref, h_ref, s_ref, d_ref):
    h = jnp.dot(x_ref[...], w_ref[...], preferred_element_type=jnp.float32)
    h_ref[...] = h
    s_ref[...] = jnp.sum(h * as_ref[...][None, :], axis=1)[None, :]
    d_ref[...] = jnp.sum(h * ad_ref[...][None, :], axis=1)[None, :]


def _tc_prep(x, W, a_s, a_d):
    return pl.pallas_call(
        _prep_body,
        out_shape=[
            jax.ShapeDtypeStruct((N, D), jnp.float32),
            jax.ShapeDtypeStruct((1, N), jnp.float32),
            jax.ShapeDtypeStruct((1, N), jnp.float32),
        ],
    )(x, W, a_s, a_d)


def _epilogue(numer_ref, den_ref):
    agg = jnp.concatenate([numer_ref[0], numer_ref[1]], axis=-1)
    den = 0.5 * jnp.sum(den_ref[...], axis=0) + 1e-9
    y = jnp.maximum(agg / den[:, None], 0.0)
    nrm = jnp.sqrt(jnp.sum(y * y, axis=1, keepdims=True))
    return y / jnp.maximum(nrm, 1e-6)


def _mid_body(numer_ref, den_ref, w_ref, as_ref, ad_ref,
              h_ref, s_ref, d_ref):
    x2 = _epilogue(numer_ref, den_ref)
    h = jnp.dot(x2, w_ref[...], preferred_element_type=jnp.float32)
    h_ref[...] = h
    s_ref[...] = jnp.sum(h * as_ref[...][None, :], axis=1)[None, :]
    d_ref[...] = jnp.sum(h * ad_ref[...][None, :], axis=1)[None, :]


def _tc_mid(numer_p, den_p, W, a_s, a_d):
    return pl.pallas_call(
        _mid_body,
        out_shape=[
            jax.ShapeDtypeStruct((N, D), jnp.float32),
            jax.ShapeDtypeStruct((1, N), jnp.float32),
            jax.ShapeDtypeStruct((1, N), jnp.float32),
        ],
    )(numer_p, den_p, W, a_s, a_d)


def _final_body(numer_ref, den_ref, out_ref):
    out_ref[...] = _epilogue(numer_ref, den_ref)


def _tc_final(numer_p, den_p):
    return pl.pallas_call(
        _final_body,
        out_shape=jax.ShapeDtypeStruct((N, D), jnp.float32),
    )(numer_p, den_p)


_MESH = plsc.VectorSubcoreMesh(core_axis_name="c", subcore_axis_name="s")


@functools.partial(
    pl.kernel,
    out_type=[
        jax.ShapeDtypeStruct((NC, N, DH), jnp.float32),
        jax.ShapeDtypeStruct((NW, N), jnp.float32),
    ],
    mesh=_MESH,
    scratch_types=[
        pltpu.VMEM((DEPTH, 2, B), jnp.int32),
        pltpu.VMEM((DEPTH, B, DH), jnp.float32),
        pltpu.VMEM((N,), jnp.float32),
        pltpu.VMEM((N,), jnp.float32),
        pltpu.VMEM((N,), jnp.float32),
        pltpu.VMEM_SHARED((N, DH), jnp.float32),
        pltpu.SemaphoreType.DMA((DEPTH,)),
        pltpu.SemaphoreType.DMA((DEPTH,)),
        pltpu.SemaphoreType.DMA((DEPTH,)),
    ],
    compiler_params=pltpu.CompilerParams(needs_layout_passes=False,
                                         use_tc_tiling_on_sc=False),
)
def _sc_edge_pass(adj_hbm, s_hbm, d_hbm, h2_hbm,
                  numer_out, den_out,
                  iring, rows, s_tab, d_tab, den_loc,
                  numer_sh, gsem, nsem, isem):
    cid = lax.axis_index("c")
    sid = lax.axis_index("s")
    gwid = cid * NS + sid
    zv = jnp.zeros((16,), jnp.float32)

    def zrow(i, _):
        for j in range(DH // 16):
            rows[0, i, pl.ds(j * 16, 16)] = zv
        return 0
    lax.fori_loop(0, B, zrow, 0)

    def zden(i, _):
        den_loc[pl.ds(i * 16, 16)] = zv
        return 0
    lax.fori_loop(0, N // 16, zden, 0)

    nchunk = N // B
    def zchunk(k, _):
        c = sid + k * NS

        @pl.when(c < nchunk)
        def _():
            pltpu.sync_copy(rows.at[0], numer_sh.at[pl.ds(c * B, B)])
        return 0
    lax.fori_loop(0, (nchunk + NS - 1) // NS, zchunk, 0)

    pltpu.sync_copy(s_hbm.at[0], s_tab)
    pltpu.sync_copy(d_hbm.at[0], d_tab)

    plsc.subcore_barrier()

    base = sid * EPT

    def ifetch_start(blk, sl):
        eb = base + blk * B
        pltpu.async_copy(adj_hbm.at[:, pl.ds(eb, B)], iring.at[sl],
                         isem.at[sl])

    def ifetch_wait(sl):
        pltpu.make_async_copy(adj_hbm.at[:, pl.ds(0, B)], iring.at[sl],
                              isem.at[sl]).wait()

    def itransform(sl):
        for j in range(B // 16):
            v = iring[sl, 0, pl.ds(j * 16, 16)]
            iring[sl, 0, pl.ds(j * 16, 16)] = v * 2 + cid

    def gather_start(sl):
        pltpu.async_copy(h2_hbm.at[iring.at[sl, 0]], rows.at[sl],
                         gsem.at[sl])

    def gather_wait(sl):
        pltpu.make_async_copy(h2_hbm.at[iring.at[sl, 0]], rows.at[sl],
                              gsem.at[sl]).wait()

    def scatter_start(sl):
        pltpu.async_copy(rows.at[sl], numer_sh.at[iring.at[sl, 1]],
                         nsem.at[sl], add=True)

    def scatter_wait(sl):
        pltpu.make_async_copy(rows.at[sl], numer_sh.at[iring.at[sl, 1]],
                              nsem.at[sl]).wait()

    def compute(sl):
        for j in range(B // 16):
            iv = iring[sl, 0, pl.ds(j * 16, 16)] // 2
            dv = iring[sl, 1, pl.ds(j * 16, 16)]
            e = plsc.load_gather(s_tab, [iv]) + plsc.load_gather(d_tab, [dv])
            e = jnp.where(e >= 0, e, 0.2 * e)
            w = jnp.exp(e)
            plsc.addupdate_scatter(den_loc, [dv], w)
            for k in range(16):
                wi = w[k]
                r = j * 16 + k
                for c in range(DH // 16):
                    rows[sl, r, pl.ds(c * 16, 16)] = (
                        rows[sl, r, pl.ds(c * 16, 16)] * wi)

    for p in range(FA):
        ifetch_start(p, p)
    for p in range(GA):
        ifetch_wait(p)
        itransform(p)
        gather_start(p)

    def group(g, _):
        for b in range(DEPTH):
            blk = g * DEPTH + b
            sl = b
            sl_g = (b + GA) % DEPTH
            sl_f = (b + FA) % DEPTH

            @pl.when(blk < NBLK)
            def _():
                @pl.when(jnp.logical_and(blk >= GA, blk + GA < NBLK))
                def _():
                    scatter_wait(sl_g)

                @pl.when(blk + FA < NBLK)
                def _():
                    ifetch_start(blk + FA, sl_f)

                @pl.when(blk + GA < NBLK)
                def _():
                    ifetch_wait(sl_g)
                    itransform(sl_g)
                    gather_start(sl_g)

                gather_wait(sl)
                compute(sl)
                scatter_start(sl)
        return 0
    lax.fori_loop(0, (NBLK + DEPTH - 1) // DEPTH, group, 0)

    for t in range(GA):
        scatter_wait((NBLK - GA + t) % DEPTH)

    pltpu.sync_copy(den_loc, den_out.at[gwid])

    plsc.subcore_barrier()

    WB = 2000

    @pl.when(sid < N // WB)
    def _():
        r0 = sid * WB
        pltpu.sync_copy(numer_sh.at[pl.ds(r0, WB)],
                        numer_out.at[cid, pl.ds(r0, WB)])


def kernel(x, adj, W1, a_src1, a_dst1, W2, a_src2, a_dst2):
    h1, s1, d1 = _tc_prep(x, W1, a_src1, a_dst1)
    numer1, den1 = _sc_edge_pass(adj[0], s1, d1, h1.reshape(2 * N, DH))
    h2, s2, d2 = _tc_mid(numer1, den1, W2, a_src2, a_dst2)
    numer2, den2 = _sc_edge_pass(adj[1], s2, d2, h2.reshape(2 * N, DH))
    return _tc_final(numer2, den2)
